# baseline (speedup 1.0000x reference)
import jax
import jax.numpy as jnp
from jax import lax
from jax.experimental import pallas as pl
from jax.experimental.pallas import tpu as pltpu
from jax.experimental.pallas import tpu_sc as plsc

N_TOK = 32768
D = 512
SEG_LEN = 2048
N_SEG = N_TOK // SEG_LEN
K_SC = 4
N_SEG_TC = N_SEG - K_SC
NC = 2
NS = 16
L = 16
NW = NC * NS
ROWS_PER_W = K_SC * SEG_LEN // NW
G = SEG_LEN // ROWS_PER_W
BLK = 64
NBLK = ROWS_PER_W // BLK
NG = D // L


def _sc_body(x_hbm, out_hbm, buf, cs_ref, peer_ref, tot_ref, sq_ref, out_v,
             shared, sem0, sem1):
    c = lax.axis_index("c")
    s = lax.axis_index("s")
    wid = c * NS + s
    base = wid * ROWS_PER_W
    sems = (sem0, sem1)

    zero = jnp.zeros((L,), jnp.float32)

    @pl.loop(0, NG)
    def _zero(j):
        cs_ref[pl.ds(j * L, L)] = zero

    sq_ref[:] = zero

    for b in range(2):
        pltpu.async_copy(x_hbm.at[pl.ds(base + b * BLK, BLK)], buf.at[b],
                         sems[b])

    @pl.loop(0, NBLK // 2)
    def _blocks(g2):
        for b in range(2):
            g = g2 * 2 + b
            pltpu.make_async_copy(x_hbm.at[pl.ds(0, BLK)], buf.at[b],
                                  sems[b]).wait()
            blk = buf.at[b]

            @pl.loop(0, NG)
            def _grp(j, blk=blk):
                jo = j * L
                def body(t8, carry, blk=blk, jo=jo):
                    a = list(carry[:4])
                    q = list(carry[4:])
                    for u in range(8):
                        v = blk[t8 * 8 + u, pl.ds(jo, L)]
                        a[u % 4] = a[u % 4] + v
                        q[u % 4] = q[u % 4] + v * v
                    return tuple(a) + tuple(q)
                r = lax.fori_loop(0, BLK // 8, body, (zero,) * 8)
                acc = (r[0] + r[1]) + (r[2] + r[3])
                sq = (r[4] + r[5]) + (r[6] + r[7])
                cs_ref[pl.ds(jo, L)] = cs_ref[pl.ds(jo, L)] + acc
                sq_ref[:] = sq_ref[:] + sq
            @pl.when(g + 2 < NBLK)
            def _refill():
                row = base + (g + 2) * BLK
                pltpu.async_copy(x_hbm.at[pl.ds(row, BLK)], buf.at[b],
                                 sems[b])

    pltpu.sync_copy(cs_ref, shared.at[s])
    plsc.subcore_barrier()
    sbase = (s // G) * G
    pltpu.sync_copy(shared.at[sbase], tot_ref)
    for m in range(1, G):
        pltpu.sync_copy(shared.at[sbase + m], peer_ref)

        @pl.loop(0, NG)
        def _tot(j):
            jo = j * L
            tot_ref[pl.ds(jo, L)] = tot_ref[pl.ds(jo, L)] + peer_ref[pl.ds(jo, L)]

    def cross(j, t_acc):
        jo = j * L
        return t_acc + cs_ref[pl.ds(jo, L)] * tot_ref[pl.ds(jo, L)]
    t_acc = lax.fori_loop(0, NG, cross, zero)
    out_v[:] = sq_ref[:] - t_acc * (1.0 / SEG_LEN)
    pltpu.sync_copy(out_v, out_hbm.at[wid])


_sc_part = pl.kernel(
    _sc_body,
    out_type=jax.ShapeDtypeStruct((NW, L), jnp.float32),
    mesh=plsc.VectorSubcoreMesh(core_axis_name="c", subcore_axis_name="s",
                                num_cores=NC, num_subcores=NS),
    scratch_types=[
        pltpu.VMEM((2, BLK, D), jnp.float32),
        pltpu.VMEM((D,), jnp.float32),
        pltpu.VMEM((D,), jnp.float32),
        pltpu.VMEM((D,), jnp.float32),
        pltpu.VMEM((L,), jnp.float32),
        pltpu.VMEM((L,), jnp.float32),
        pltpu.VMEM_SHARED((NS, D), jnp.float32),
        pltpu.SemaphoreType.DMA,
        pltpu.SemaphoreType.DMA,
    ],
)


SUB = 4
TC_ROWS = SEG_LEN // SUB


def _tc_body(x_ref, out_ref, cs_ref):
    i = pl.program_id(0)
    sub = i % SUB
    x = x_ref[...]
    csb = jnp.sum(x, axis=0, keepdims=True)
    sq = jnp.sum(x * x)

    @pl.when(i == 0)
    def _init():
        out_ref[...] = jnp.zeros_like(out_ref)

    @pl.when(sub == 0)
    def _cs_init():
        cs_ref[...] = csb

    @pl.when(sub != 0)
    def _cs_acc():
        cs_ref[...] += csb

    out_ref[...] += sq

    @pl.when(sub == SUB - 1)
    def _fold():
        c = cs_ref[...]
        out_ref[...] += -jnp.sum(c * c) * (1.0 / SEG_LEN)


_tc_part = pl.pallas_call(
    _tc_body,
    grid=(N_SEG_TC * SUB,),
    in_specs=[pl.BlockSpec((TC_ROWS, D), lambda i: (i + K_SC * SUB, 0))],
    out_specs=pl.BlockSpec((1, 128), lambda i: (0, 0)),
    out_shape=jax.ShapeDtypeStruct((1, 128), jnp.float32),
    scratch_shapes=[pltpu.VMEM((1, D), jnp.float32)],
)


def kernel(inputs, lengths):
    del lengths
    sc = _sc_part(inputs)
    tc = _tc_part(inputs)
    return (jnp.sum(sc) + tc[0, 0]) / (N_TOK * D)


# speedup vs baseline: 1.5304x; 1.5304x over previous
import jax
import jax.numpy as jnp
from jax import lax
from jax.experimental import pallas as pl
from jax.experimental.pallas import tpu as pltpu
from jax.experimental.pallas import tpu_sc as plsc

N_TOK = 32768
D = 512
SEG_LEN = 2048
N_SEG = N_TOK // SEG_LEN
K_SC = 4
N_SEG_TC = N_SEG - K_SC
NC = 2
NS = 16
L = 16
NW = NC * NS
ROWS_PER_W = K_SC * SEG_LEN // NW
G = SEG_LEN // ROWS_PER_W
BLK = 64
NBLK = ROWS_PER_W // BLK
NG = D // L


def _sc_body(x_hbm, out_hbm, buf, cs_ref, peer_ref, tot_ref, sq_ref, out_v,
             shared, sem0, sem1):
    c = lax.axis_index("c")
    s = lax.axis_index("s")
    wid = c * NS + s
    base = wid * ROWS_PER_W
    sems = (sem0, sem1)

    zero = jnp.zeros((L,), jnp.float32)

    @pl.loop(0, NG)
    def _zero(j):
        cs_ref[pl.ds(j * L, L)] = zero

    sq_ref[:] = zero

    for b in range(2):
        pltpu.async_copy(x_hbm.at[pl.ds(base + b * BLK, BLK)], buf.at[b],
                         sems[b])

    @pl.loop(0, NBLK // 2)
    def _blocks(g2):
        for b in range(2):
            g = g2 * 2 + b
            pltpu.make_async_copy(x_hbm.at[pl.ds(0, BLK)], buf.at[b],
                                  sems[b]).wait()
            blk = buf.at[b]

            @pl.loop(0, NG)
            def _grp(j, blk=blk):
                jo = j * L
                def body(t8, carry, blk=blk, jo=jo):
                    a = list(carry[:4])
                    q = list(carry[4:])
                    for u in range(8):
                        v = blk[t8 * 8 + u, pl.ds(jo, L)]
                        a[u % 4] = a[u % 4] + v
                        q[u % 4] = q[u % 4] + v * v
                    return tuple(a) + tuple(q)
                r = lax.fori_loop(0, BLK // 8, body, (zero,) * 8)
                acc = (r[0] + r[1]) + (r[2] + r[3])
                sq = (r[4] + r[5]) + (r[6] + r[7])
                cs_ref[pl.ds(jo, L)] = cs_ref[pl.ds(jo, L)] + acc
                sq_ref[:] = sq_ref[:] + sq
            @pl.when(g + 2 < NBLK)
            def _refill():
                row = base + (g + 2) * BLK
                pltpu.async_copy(x_hbm.at[pl.ds(row, BLK)], buf.at[b],
                                 sems[b])

    pltpu.sync_copy(cs_ref, shared.at[s])
    plsc.subcore_barrier()
    sbase = (s // G) * G
    pltpu.sync_copy(shared.at[sbase], tot_ref)
    for m in range(1, G):
        pltpu.sync_copy(shared.at[sbase + m], peer_ref)

        @pl.loop(0, NG)
        def _tot(j):
            jo = j * L
            tot_ref[pl.ds(jo, L)] = tot_ref[pl.ds(jo, L)] + peer_ref[pl.ds(jo, L)]

    def cross(j, t_acc):
        jo = j * L
        return t_acc + cs_ref[pl.ds(jo, L)] * tot_ref[pl.ds(jo, L)]
    t_acc = lax.fori_loop(0, NG, cross, zero)
    out_v[:] = sq_ref[:] - t_acc * (1.0 / SEG_LEN)
    pltpu.sync_copy(out_v, out_hbm.at[wid])


_sc_part = pl.kernel(
    _sc_body,
    out_type=jax.ShapeDtypeStruct((NW, L), jnp.float32),
    mesh=plsc.VectorSubcoreMesh(core_axis_name="c", subcore_axis_name="s",
                                num_cores=NC, num_subcores=NS),
    scratch_types=[
        pltpu.VMEM((2, BLK, D), jnp.float32),
        pltpu.VMEM((D,), jnp.float32),
        pltpu.VMEM((D,), jnp.float32),
        pltpu.VMEM((D,), jnp.float32),
        pltpu.VMEM((L,), jnp.float32),
        pltpu.VMEM((L,), jnp.float32),
        pltpu.VMEM_SHARED((NS, D), jnp.float32),
        pltpu.SemaphoreType.DMA,
        pltpu.SemaphoreType.DMA,
    ],
)


def _tc_body(x_ref, out_ref, sqv_ref, csq_ref):
    i = pl.program_id(0)
    x = x_ref[...]
    cs = jnp.sum(x, axis=0, keepdims=True)
    sqb = jnp.sum(x * x, axis=0, keepdims=True)

    @pl.when(i == 0)
    def _init():
        sqv_ref[...] = sqb
        csq_ref[...] = cs * cs

    @pl.when(i != 0)
    def _acc():
        sqv_ref[...] += sqb
        csq_ref[...] += cs * cs

    @pl.when(i == N_SEG_TC - 1)
    def _fold():
        out_ref[...] = jnp.full(
            (1, 128),
            jnp.sum(sqv_ref[...] - csq_ref[...] * (1.0 / SEG_LEN)),
            jnp.float32)


_tc_part = pl.pallas_call(
    _tc_body,
    grid=(N_SEG_TC,),
    in_specs=[pl.BlockSpec((SEG_LEN, D), lambda i: (i + K_SC, 0))],
    out_specs=pl.BlockSpec((1, 128), lambda i: (0, 0)),
    out_shape=jax.ShapeDtypeStruct((1, 128), jnp.float32),
    scratch_shapes=[pltpu.VMEM((1, D), jnp.float32),
                    pltpu.VMEM((1, D), jnp.float32)],
)


def kernel(inputs, lengths):
    del lengths
    sc = _sc_part(inputs)
    tc = _tc_part(inputs)
    return (jnp.sum(sc) + tc[0, 0]) / (N_TOK * D)


# speedup vs baseline: 1.5363x; 1.0038x over previous
import jax
import jax.numpy as jnp
from jax import lax
from jax.experimental import pallas as pl
from jax.experimental.pallas import tpu as pltpu
from jax.experimental.pallas import tpu_sc as plsc

N_TOK = 32768
D = 512
SEG_LEN = 2048
N_SEG = N_TOK // SEG_LEN
K_SC = 4
N_SEG_TC = N_SEG - K_SC
NC = 2
NS = 16
L = 16
NW = NC * NS
ROWS_PER_W = K_SC * SEG_LEN // NW
G = SEG_LEN // ROWS_PER_W
BLK = 64
NBLK = ROWS_PER_W // BLK
NG = D // L


def _sc_body(x_hbm, out_hbm, buf, cs_ref, peer_ref, tot_ref, sq_ref, out_v,
             parts_ref, shared, shared2, sem0, sem1):
    c = lax.axis_index("c")
    s = lax.axis_index("s")
    wid = c * NS + s
    base = wid * ROWS_PER_W
    sems = (sem0, sem1)

    zero = jnp.zeros((L,), jnp.float32)

    @pl.loop(0, NG)
    def _zero(j):
        cs_ref[pl.ds(j * L, L)] = zero

    sq_ref[:] = zero

    for b in range(2):
        pltpu.async_copy(x_hbm.at[pl.ds(base + b * BLK, BLK)], buf.at[b],
                         sems[b])

    @pl.loop(0, NBLK // 2)
    def _blocks(g2):
        for b in range(2):
            g = g2 * 2 + b
            pltpu.make_async_copy(x_hbm.at[pl.ds(0, BLK)], buf.at[b],
                                  sems[b]).wait()
            blk = buf.at[b]

            @pl.loop(0, NG)
            def _grp(j, blk=blk):
                jo = j * L
                def body(t8, carry, blk=blk, jo=jo):
                    a = list(carry[:4])
                    q = list(carry[4:])
                    for u in range(8):
                        v = blk[t8 * 8 + u, pl.ds(jo, L)]
                        a[u % 4] = a[u % 4] + v
                        q[u % 4] = q[u % 4] + v * v
                    return tuple(a) + tuple(q)
                r = lax.fori_loop(0, BLK // 8, body, (zero,) * 8)
                acc = (r[0] + r[1]) + (r[2] + r[3])
                sq = (r[4] + r[5]) + (r[6] + r[7])
                cs_ref[pl.ds(jo, L)] = cs_ref[pl.ds(jo, L)] + acc
                sq_ref[:] = sq_ref[:] + sq
            @pl.when(g + 2 < NBLK)
            def _refill():
                row = base + (g + 2) * BLK
                pltpu.async_copy(x_hbm.at[pl.ds(row, BLK)], buf.at[b],
                                 sems[b])

    pltpu.sync_copy(cs_ref, shared.at[s])
    plsc.subcore_barrier()
    sbase = (s // G) * G
    pltpu.sync_copy(shared.at[sbase], tot_ref)
    for m in range(1, G):
        pltpu.sync_copy(shared.at[sbase + m], peer_ref)

        @pl.loop(0, NG)
        def _tot(j):
            jo = j * L
            tot_ref[pl.ds(jo, L)] = tot_ref[pl.ds(jo, L)] + peer_ref[pl.ds(jo, L)]

    def cross(j, t_acc):
        jo = j * L
        return t_acc + cs_ref[pl.ds(jo, L)] * tot_ref[pl.ds(jo, L)]
    t_acc = lax.fori_loop(0, NG, cross, zero)
    out_v[:] = sq_ref[:] - t_acc * (1.0 / SEG_LEN)

    pltpu.sync_copy(out_v, shared2.at[s])
    plsc.subcore_barrier()

    @pl.when(s == 0)
    def _final():
        pltpu.sync_copy(shared2, parts_ref)
        acc = parts_ref[0, :]
        for m in range(1, NS):
            acc = acc + parts_ref[m, :]
        out_v[:] = acc
        pltpu.sync_copy(out_v, out_hbm.at[c])


_sc_part = pl.kernel(
    _sc_body,
    out_type=jax.ShapeDtypeStruct((NC, L), jnp.float32),
    mesh=plsc.VectorSubcoreMesh(core_axis_name="c", subcore_axis_name="s",
                                num_cores=NC, num_subcores=NS),
    scratch_types=[
        pltpu.VMEM((2, BLK, D), jnp.float32),
        pltpu.VMEM((D,), jnp.float32),
        pltpu.VMEM((D,), jnp.float32),
        pltpu.VMEM((D,), jnp.float32),
        pltpu.VMEM((L,), jnp.float32),
        pltpu.VMEM((L,), jnp.float32),
        pltpu.VMEM((NS, L), jnp.float32),
        pltpu.VMEM_SHARED((NS, D), jnp.float32),
        pltpu.VMEM_SHARED((NS, L), jnp.float32),
        pltpu.SemaphoreType.DMA,
        pltpu.SemaphoreType.DMA,
    ],
)


def _tc_body(x_ref, out_ref, sqv_ref, csq_ref):
    i = pl.program_id(0)
    x = x_ref[...]
    cs = jnp.sum(x, axis=0, keepdims=True)
    sqb = jnp.sum(x * x, axis=0, keepdims=True)

    @pl.when(i == 0)
    def _init():
        sqv_ref[...] = sqb
        csq_ref[...] = cs * cs

    @pl.when(i != 0)
    def _acc():
        sqv_ref[...] += sqb
        csq_ref[...] += cs * cs

    @pl.when(i == N_SEG_TC - 1)
    def _fold():
        out_ref[...] = jnp.full(
            (1, 128),
            jnp.sum(sqv_ref[...] - csq_ref[...] * (1.0 / SEG_LEN)),
            jnp.float32)


_tc_part = pl.pallas_call(
    _tc_body,
    grid=(N_SEG_TC,),
    in_specs=[pl.BlockSpec((SEG_LEN, D), lambda i: (i + K_SC, 0))],
    out_specs=pl.BlockSpec((1, 128), lambda i: (0, 0)),
    out_shape=jax.ShapeDtypeStruct((1, 128), jnp.float32),
    scratch_shapes=[pltpu.VMEM((1, D), jnp.float32),
                    pltpu.VMEM((1, D), jnp.float32)],
)


def kernel(inputs, lengths):
    del lengths
    sc = _sc_part(inputs)
    tc = _tc_part(inputs)
    return (jnp.sum(sc) + tc[0, 0]) / (N_TOK * D)


# speedup vs baseline: 1.5452x; 1.0058x over previous
import jax
import jax.numpy as jnp
from jax import lax
from jax.experimental import pallas as pl
from jax.experimental.pallas import tpu as pltpu
from jax.experimental.pallas import tpu_sc as plsc

N_TOK = 32768
D = 512
SEG_LEN = 2048
N_SEG = N_TOK // SEG_LEN
K_SC = 4
N_SEG_TC = N_SEG - K_SC
NC = 2
NS = 16
L = 16
NW = NC * NS
ROWS_PER_W = K_SC * SEG_LEN // NW
G = SEG_LEN // ROWS_PER_W
BLK = 64
NBLK = ROWS_PER_W // BLK
NG = D // L


def _sc_body(x_hbm, out_hbm, buf, cs_ref, peer_ref, tot_ref, sq_ref, out_v,
             shared, sem0, sem1):
    c = lax.axis_index("c")
    s = lax.axis_index("s")
    wid = c * NS + s
    base = wid * ROWS_PER_W
    sems = (sem0, sem1)

    zero = jnp.zeros((L,), jnp.float32)

    @pl.loop(0, NG)
    def _zero(j):
        cs_ref[pl.ds(j * L, L)] = zero

    sq_ref[:] = zero

    for b in range(2):
        pltpu.async_copy(x_hbm.at[pl.ds(base + b * BLK, BLK)], buf.at[b],
                         sems[b])

    @pl.loop(0, NBLK // 2)
    def _blocks(g2):
        for b in range(2):
            g = g2 * 2 + b
            pltpu.make_async_copy(x_hbm.at[pl.ds(0, BLK)], buf.at[b],
                                  sems[b]).wait()
            blk = buf.at[b]

            @pl.loop(0, NG)
            def _grp(j, blk=blk):
                jo = j * L
                def body(t8, carry, blk=blk, jo=jo):
                    a = list(carry[:4])
                    q = list(carry[4:])
                    for u in range(8):
                        v = blk[t8 * 8 + u, pl.ds(jo, L)]
                        a[u % 4] = a[u % 4] + v
                        q[u % 4] = q[u % 4] + v * v
                    return tuple(a) + tuple(q)
                r = lax.fori_loop(0, BLK // 8, body, (zero,) * 8)
                acc = (r[0] + r[1]) + (r[2] + r[3])
                sq = (r[4] + r[5]) + (r[6] + r[7])
                cs_ref[pl.ds(jo, L)] = cs_ref[pl.ds(jo, L)] + acc
                sq_ref[:] = sq_ref[:] + sq
            @pl.when(g + 2 < NBLK)
            def _refill():
                row = base + (g + 2) * BLK
                pltpu.async_copy(x_hbm.at[pl.ds(row, BLK)], buf.at[b],
                                 sems[b])

    pltpu.sync_copy(cs_ref, shared.at[s])
    plsc.subcore_barrier()
    sbase = (s // G) * G
    pltpu.sync_copy(shared.at[sbase], tot_ref)
    for m in range(1, G):
        pltpu.sync_copy(shared.at[sbase + m], peer_ref)

        @pl.loop(0, NG)
        def _tot(j):
            jo = j * L
            tot_ref[pl.ds(jo, L)] = tot_ref[pl.ds(jo, L)] + peer_ref[pl.ds(jo, L)]

    def cross(j, t_acc):
        jo = j * L
        return t_acc + cs_ref[pl.ds(jo, L)] * tot_ref[pl.ds(jo, L)]
    t_acc = lax.fori_loop(0, NG, cross, zero)
    out_v[:] = sq_ref[:] - t_acc * (1.0 / SEG_LEN)
    pltpu.sync_copy(out_v, out_hbm.at[wid])


_sc_part = pl.kernel(
    _sc_body,
    out_type=jax.ShapeDtypeStruct((NW, L), jnp.float32),
    mesh=plsc.VectorSubcoreMesh(core_axis_name="c", subcore_axis_name="s",
                                num_cores=NC, num_subcores=NS),
    scratch_types=[
        pltpu.VMEM((2, BLK, D), jnp.float32),
        pltpu.VMEM((D,), jnp.float32),
        pltpu.VMEM((D,), jnp.float32),
        pltpu.VMEM((D,), jnp.float32),
        pltpu.VMEM((L,), jnp.float32),
        pltpu.VMEM((L,), jnp.float32),
        pltpu.VMEM_SHARED((NS, D), jnp.float32),
        pltpu.SemaphoreType.DMA,
        pltpu.SemaphoreType.DMA,
    ],
)


def _tc_body(x_ref, out_ref, sqv_ref, csq_ref):
    i = pl.program_id(0)
    x = x_ref[...]
    cs = jnp.sum(x, axis=0, keepdims=True)
    sqb = jnp.sum(x * x, axis=0, keepdims=True)

    @pl.when(i == 0)
    def _init():
        sqv_ref[...] = sqb
        csq_ref[...] = cs * cs

    @pl.when(i != 0)
    def _acc():
        sqv_ref[...] += sqb
        csq_ref[...] += cs * cs

    @pl.when(i == N_SEG_TC - 1)
    def _fold():
        out_ref[...] = jnp.full(
            (1, 128),
            jnp.sum(sqv_ref[...] - csq_ref[...] * (1.0 / SEG_LEN)),
            jnp.float32)


_tc_part = pl.pallas_call(
    _tc_body,
    grid=(N_SEG_TC,),
    in_specs=[pl.BlockSpec((SEG_LEN, D), lambda i: (i + K_SC, 0))],
    out_specs=pl.BlockSpec((1, 128), lambda i: (0, 0)),
    out_shape=jax.ShapeDtypeStruct((1, 128), jnp.float32),
    scratch_shapes=[pltpu.VMEM((1, D), jnp.float32),
                    pltpu.VMEM((1, D), jnp.float32)],
)


def kernel(inputs, lengths):
    del lengths
    tc = _tc_part(inputs)
    sc = _sc_part(inputs)
    return (jnp.sum(sc) + tc[0, 0]) / (N_TOK * D)
